# Initial kernel scaffold; baseline (speedup 1.0000x reference)
import jax
import jax.numpy as jnp
from jax.experimental import pallas as pl


def kernel(input_):
    raise NotImplementedError("write your pallas kernel here")


# speedup vs baseline: 1.3824x; 1.3824x over previous
import jax
import jax.numpy as jnp
from jax import lax
from jax.experimental import pallas as pl
from jax.experimental.pallas import tpu as pltpu

K = 64
LANE = 128


def _row_kernel(x_ref, o_ref):
    x = x_ref[0]
    sub = x.shape[0]
    bits = lax.bitcast_convert_type(x, jnp.int32) & jnp.int32(0x7FFFFFFF)

    def body(i, t):
        cand = t | (jnp.int32(1) << (30 - i))
        cnt = jnp.sum((bits >= cand).astype(jnp.int32))
        return lax.select(cnt >= K, cand, t)

    t = lax.fori_loop(0, 31, body, jnp.int32(0))

    gt = bits > t
    eq = bits == t
    c_gt = jnp.sum(gt.astype(jnp.int32))
    quota = (K - c_gt).astype(jnp.float32)

    eqf = eq.astype(jnp.float32)
    tri_l = (lax.broadcasted_iota(jnp.int32, (LANE, LANE), 0)
             <= lax.broadcasted_iota(jnp.int32, (LANE, LANE), 1)
             ).astype(jnp.float32)
    psum = jnp.dot(eqf, tri_l, preferred_element_type=jnp.float32)
    tot = psum[:, LANE - 1:LANE]
    tri_s = (lax.broadcasted_iota(jnp.int32, (sub, sub), 1)
             < lax.broadcasted_iota(jnp.int32, (sub, sub), 0)
             ).astype(jnp.float32)
    offs = jnp.dot(tri_s, tot, preferred_element_type=jnp.float32)
    rank = psum + offs

    keep = gt | (eq & (rank <= quota))
    o_ref[0] = jnp.where(keep, x, jnp.float32(0.0))


def kernel(input_):
    r, c = input_.shape
    sub = c // LANE
    x3 = input_.reshape(r, sub, LANE)
    out = pl.pallas_call(
        _row_kernel,
        grid=(r,),
        in_specs=[pl.BlockSpec((1, sub, LANE), lambda i: (i, 0, 0))],
        out_specs=pl.BlockSpec((1, sub, LANE), lambda i: (i, 0, 0)),
        out_shape=jax.ShapeDtypeStruct((r, sub, LANE), jnp.float32),
        compiler_params=pltpu.CompilerParams(
            dimension_semantics=("arbitrary",),
        ),
    )(x3)
    return out.reshape(r, c)


# speedup vs baseline: 2.9453x; 2.1306x over previous
import functools

import jax
import jax.numpy as jnp
from jax import lax
from jax.experimental import pallas as pl
from jax.experimental.pallas import tpu as pltpu
from jax.experimental.pallas import tpu_sc as plsc

K = 64
R, C = 128, 32768
L = 16
NSL = C // L
NW = 32
ROWS_PER_W = R // NW
CAND2_CAP = 8192
SIGN = 0x7FFFFFFF


def _lane():
    return lax.broadcasted_iota(jnp.int32, (L,), 0)


def _clear(ref, nslices):
    def body(i, c):
        ref[pl.ds(i * L, L)] = jnp.zeros((L,), jnp.int32)
        return c
    lax.fori_loop(0, nslices, body, 0)


def _select_bucket(hist, nb, k):
    def sfx(j, acc):
        b = nb - 1 - j
        acc = acc + hist[pl.ds(b * L, L)]
        hist[pl.ds(b * L, L)] = acc
        return acc
    lax.fori_loop(0, nb, sfx, jnp.zeros((L,), jnp.int32))

    def stot(b):
        return jnp.sum(hist[pl.ds(b * L, L)])

    def bit(i, b):
        cand = b | (jnp.int32(1) << (7 - i))
        ok = (cand < nb) & (stot(cand) >= k)
        return lax.select(ok, cand, b)
    b = lax.fori_loop(0, 8, bit, jnp.int32(0))
    above = stot(b + 1)
    cnt_at = stot(b) - above
    return b, above, cnt_at


def _hist_row(row, hist):
    lane = _lane()
    ones = jnp.ones((L,), jnp.int32)

    def body(i, c):
        v = row[pl.ds(i * L, L)]
        bits = lax.bitcast_convert_type(v, jnp.int32) & SIGN
        idx = ((bits >> 23) << 4) | lane
        plsc.addupdate_scatter(hist, [idx], ones)
        return c
    lax.fori_loop(0, NSL, body, 0)


def _compact_row(row, dst, bsel):
    lane = _lane()

    def body(i, off):
        v = row[pl.ds(i * L, L)]
        bits = lax.bitcast_convert_type(v, jnp.int32) & SIGN
        m = (bits >> 23) == bsel
        pos = plsc.cumsum(m.astype(jnp.int32))
        plsc.store_scatter(dst, [off + pos - 1], bits, mask=m)
        return off + plsc.all_reduce_population_count(m)
    off = lax.fori_loop(0, NSL, body, jnp.zeros((L,), jnp.int32))
    return jnp.squeeze(lax.slice(off, (0,), (1,)))


def _hist_cand(src, n, shift, bmask, hist):
    lane = _lane()
    ones = jnp.ones((L,), jnp.int32)
    nsl = (n + L - 1) // L

    def body(i, c):
        valid = lane < (n - i * L)
        bits = src[pl.ds(i * L, L)]
        idx = (((bits >> shift) & bmask) << 4) | lane
        plsc.addupdate_scatter(hist, [idx], ones, mask=valid)
        return c
    lax.fori_loop(0, nsl, body, 0)


def _compact_cand(src, dst, n, shift, bmask, bsel, cap):
    lane = _lane()
    nsl = (n + L - 1) // L

    def body(i, off):
        valid = lane < (n - i * L)
        bits = src[pl.ds(i * L, L)]
        m = valid & (((bits >> shift) & bmask) == bsel)
        pos = plsc.cumsum(m.astype(jnp.int32))
        idx = jnp.minimum(off + pos - 1, jnp.int32(cap - 1))
        plsc.store_scatter(dst, [idx], bits, mask=m)
        return off + plsc.all_reduce_population_count(m)
    off = lax.fori_loop(0, nsl, body, jnp.zeros((L,), jnp.int32))
    return jnp.squeeze(lax.slice(off, (0,), (1,)))


def _mask_row(row, t, quota, tie_partial):
    zero = jnp.zeros((L,), jnp.float32)

    @pl.when(jnp.logical_not(tie_partial))
    def _():
        def body(i, c):
            v = row[pl.ds(i * L, L)]
            bits = lax.bitcast_convert_type(v, jnp.int32) & SIGN
            row[pl.ds(i * L, L)] = jnp.where(bits >= t, v, zero)
            return c
        lax.fori_loop(0, NSL, body, 0)

    @pl.when(tie_partial)
    def _():
        def body(i, base):
            v = row[pl.ds(i * L, L)]
            bits = lax.bitcast_convert_type(v, jnp.int32) & SIGN
            eq = bits == t
            rank = base + plsc.cumsum(eq.astype(jnp.int32))
            keep = (bits > t) | (eq & (rank <= quota))
            row[pl.ds(i * L, L)] = jnp.where(keep, v, zero)
            return base + plsc.all_reduce_population_count(eq)
        lax.fori_loop(0, NSL, body, jnp.zeros((L,), jnp.int32))


_MESH = plsc.VectorSubcoreMesh(core_axis_name="c", subcore_axis_name="s")


@functools.partial(
    pl.kernel,
    mesh=_MESH,
    out_type=jax.ShapeDtypeStruct((R, C), jnp.float32),
    compiler_params=pltpu.CompilerParams(needs_layout_passes=False),
    scratch_types=[
        pltpu.VMEM((C,), jnp.float32),
        pltpu.VMEM((4224,), jnp.int32),
        pltpu.VMEM((C,), jnp.int32),
        pltpu.VMEM((CAND2_CAP,), jnp.int32),
    ],
)
def _sc_topk(x_hbm, out_hbm, row_v, hist_v, cand_v, cand2_v):
    wid = lax.axis_index("s") * 2 + lax.axis_index("c")

    def per_row(ri, c):
        r = wid * ROWS_PER_W + ri
        pltpu.sync_copy(x_hbm.at[r], row_v)

        _clear(hist_v, 257)
        _hist_row(row_v, hist_v)
        b1, above1, cnt1 = _select_bucket(hist_v, 256, jnp.int32(K))
        k1 = jnp.int32(K) - above1
        m1 = _compact_row(row_v, cand_v, b1)

        _clear(hist_v, 257)
        _hist_cand(cand_v, m1, 15, jnp.int32(0xFF), hist_v)
        b2, above2, cnt2 = _select_bucket(hist_v, 256, k1)
        k2 = k1 - above2
        m2 = _compact_cand(cand_v, cand2_v, m1, 15, jnp.int32(0xFF), b2,
                           CAND2_CAP)

        _clear(hist_v, 257)
        _hist_cand(cand2_v, m2, 7, jnp.int32(0xFF), hist_v)
        b3, above3, cnt3 = _select_bucket(hist_v, 256, k2)
        k3 = k2 - above3
        m3 = _compact_cand(cand2_v, cand_v, m2, 7, jnp.int32(0xFF), b3, C)

        _clear(hist_v, 129)
        _hist_cand(cand_v, m3, 0, jnp.int32(0x7F), hist_v)
        b4, above4, cnt4 = _select_bucket(hist_v, 128, k3)
        quota = k3 - above4

        t = (b1 << 23) | (b2 << 15) | (b3 << 7) | b4
        _mask_row(row_v, t, quota, quota < cnt4)
        pltpu.sync_copy(row_v, out_hbm.at[r])
        return c

    lax.fori_loop(0, ROWS_PER_W, per_row, 0)


def kernel(input_):
    return _sc_topk(input_)


# speedup vs baseline: 3.8096x; 1.2934x over previous
import functools

import jax
import jax.numpy as jnp
from jax import lax
from jax.experimental import pallas as pl
from jax.experimental.pallas import tpu as pltpu
from jax.experimental.pallas import tpu_sc as plsc

K = 64
R, C = 128, 32768
L = 16
NSL = C // L
NW = 32
ROWS_PER_W = R // NW
CAND2_CAP = 8192
SIGN = 0x7FFFFFFF


def _lane():
    return lax.broadcasted_iota(jnp.int32, (L,), 0)


def _clear(ref, nslices):
    def body(i, c):
        base = i * (L * 8)
        for j in range(8):
            ref[pl.ds(base + j * L, L)] = jnp.zeros((L,), jnp.int32)
        return c
    lax.fori_loop(0, (nslices + 7) // 8, body, 0)


def _select_bucket(hist, nb, k):
    def sfx(j, acc):
        b = nb - 1 - j
        acc = acc + hist[pl.ds(b * L, L)]
        hist[pl.ds(b * L, L)] = acc
        return acc
    lax.fori_loop(0, nb, sfx, jnp.zeros((L,), jnp.int32))

    def stot(b):
        return jnp.sum(hist[pl.ds(b * L, L)])

    def bit(i, b):
        cand = b | (jnp.int32(1) << (7 - i))
        ok = (cand < nb) & (stot(cand) >= k)
        return lax.select(ok, cand, b)
    b = lax.fori_loop(0, 8, bit, jnp.int32(0))
    above = stot(b + 1)
    cnt_at = stot(b) - above
    return b, above, cnt_at


UNROLL = 8


def _hist_row(row, hist):
    lane = _lane()
    ones = jnp.ones((L,), jnp.int32)

    def body(i, c):
        base = i * (L * UNROLL)
        for j in range(UNROLL):
            v = row[pl.ds(base + j * L, L)]
            bits = lax.bitcast_convert_type(v, jnp.int32) & SIGN
            idx = ((bits >> 23) << 4) | lane
            plsc.addupdate_scatter(hist, [idx], ones)
        return c
    lax.fori_loop(0, NSL // UNROLL, body, 0)


def _compact_row(row, dst, bsel):
    lane = _lane()

    def body(i, off):
        base = i * (L * UNROLL)
        for j in range(UNROLL):
            v = row[pl.ds(base + j * L, L)]
            bits = lax.bitcast_convert_type(v, jnp.int32) & SIGN
            m = (bits >> 23) == bsel
            pos = plsc.cumsum(m.astype(jnp.int32))
            plsc.store_scatter(dst, [off + pos - 1], bits, mask=m)
            off = off + plsc.all_reduce_population_count(m)
        return off
    off = lax.fori_loop(0, NSL // UNROLL, body, jnp.zeros((L,), jnp.int32))
    return jnp.squeeze(lax.slice(off, (0,), (1,)))


def _hist_cand(src, n, shift, bmask, hist):
    lane = _lane()
    ones = jnp.ones((L,), jnp.int32)
    nsl = (n + L - 1) // L

    def body(i, c):
        valid = lane < (n - i * L)
        bits = src[pl.ds(i * L, L)]
        idx = (((bits >> shift) & bmask) << 4) | lane
        plsc.addupdate_scatter(hist, [idx], ones, mask=valid)
        return c
    lax.fori_loop(0, nsl, body, 0)


def _compact_cand(src, dst, n, shift, bmask, bsel, cap):
    lane = _lane()
    nsl = (n + L - 1) // L

    def body(i, off):
        valid = lane < (n - i * L)
        bits = src[pl.ds(i * L, L)]
        m = valid & (((bits >> shift) & bmask) == bsel)
        pos = plsc.cumsum(m.astype(jnp.int32))
        idx = jnp.minimum(off + pos - 1, jnp.int32(cap - 1))
        plsc.store_scatter(dst, [idx], bits, mask=m)
        return off + plsc.all_reduce_population_count(m)
    off = lax.fori_loop(0, nsl, body, jnp.zeros((L,), jnp.int32))
    return jnp.squeeze(lax.slice(off, (0,), (1,)))


def _mask_row(row, t, quota, tie_partial):
    zero = jnp.zeros((L,), jnp.float32)

    @pl.when(jnp.logical_not(tie_partial))
    def _():
        def body(i, c):
            base = i * (L * UNROLL)
            for j in range(UNROLL):
                v = row[pl.ds(base + j * L, L)]
                bits = lax.bitcast_convert_type(v, jnp.int32) & SIGN
                row[pl.ds(base + j * L, L)] = jnp.where(bits >= t, v, zero)
            return c
        lax.fori_loop(0, NSL // UNROLL, body, 0)

    @pl.when(tie_partial)
    def _():
        def body(i, base):
            v = row[pl.ds(i * L, L)]
            bits = lax.bitcast_convert_type(v, jnp.int32) & SIGN
            eq = bits == t
            rank = base + plsc.cumsum(eq.astype(jnp.int32))
            keep = (bits > t) | (eq & (rank <= quota))
            row[pl.ds(i * L, L)] = jnp.where(keep, v, zero)
            return base + plsc.all_reduce_population_count(eq)
        lax.fori_loop(0, NSL, body, jnp.zeros((L,), jnp.int32))


_MESH = plsc.VectorSubcoreMesh(core_axis_name="c", subcore_axis_name="s")


@functools.partial(
    pl.kernel,
    mesh=_MESH,
    out_type=jax.ShapeDtypeStruct((R, C), jnp.float32),
    compiler_params=pltpu.CompilerParams(needs_layout_passes=False),
    scratch_types=[
        pltpu.VMEM((C,), jnp.float32),
        pltpu.VMEM((4224,), jnp.int32),
        pltpu.VMEM((C,), jnp.int32),
        pltpu.VMEM((CAND2_CAP,), jnp.int32),
    ],
)
def _sc_topk(x_hbm, out_hbm, row_v, hist_v, cand_v, cand2_v):
    wid = lax.axis_index("s") * 2 + lax.axis_index("c")

    def per_row(ri, c):
        r = wid * ROWS_PER_W + ri
        pltpu.sync_copy(x_hbm.at[r], row_v)

        _clear(hist_v, 257)
        _hist_row(row_v, hist_v)
        b1, above1, cnt1 = _select_bucket(hist_v, 256, jnp.int32(K))
        k1 = jnp.int32(K) - above1
        m1 = _compact_row(row_v, cand_v, b1)

        _clear(hist_v, 257)
        _hist_cand(cand_v, m1, 15, jnp.int32(0xFF), hist_v)
        b2, above2, cnt2 = _select_bucket(hist_v, 256, k1)
        k2 = k1 - above2
        m2 = _compact_cand(cand_v, cand2_v, m1, 15, jnp.int32(0xFF), b2,
                           CAND2_CAP)

        _clear(hist_v, 257)
        _hist_cand(cand2_v, m2, 7, jnp.int32(0xFF), hist_v)
        b3, above3, cnt3 = _select_bucket(hist_v, 256, k2)
        k3 = k2 - above3
        m3 = _compact_cand(cand2_v, cand_v, m2, 7, jnp.int32(0xFF), b3, C)

        _clear(hist_v, 129)
        _hist_cand(cand_v, m3, 0, jnp.int32(0x7F), hist_v)
        b4, above4, cnt4 = _select_bucket(hist_v, 128, k3)
        quota = k3 - above4

        t = (b1 << 23) | (b2 << 15) | (b3 << 7) | b4
        _mask_row(row_v, t, quota, quota < cnt4)
        pltpu.sync_copy(row_v, out_hbm.at[r])
        return c

    lax.fori_loop(0, ROWS_PER_W, per_row, 0)


def kernel(input_):
    return _sc_topk(input_)


# speedup vs baseline: 11.1629x; 2.9302x over previous
import functools

import jax
import jax.numpy as jnp
from jax import lax
from jax.experimental import pallas as pl
from jax.experimental.pallas import tpu as pltpu
from jax.experimental.pallas import tpu_sc as plsc

K = 64
R, C = 128, 32768
L = 16
NSL = C // L
NW = 32
ROWS_PER_W = R // NW
CAND2_CAP = 8192
SIGN = 0x7FFFFFFF


def _lane():
    return lax.broadcasted_iota(jnp.int32, (L,), 0)


def _clear(ref, nslices):
    def body(i, c):
        base = i * (L * 8)
        for j in range(8):
            ref[pl.ds(base + j * L, L)] = jnp.zeros((L,), jnp.int32)
        return c
    lax.fori_loop(0, (nslices + 7) // 8, body, 0)


def _select_bucket(hist, nb, k):
    def sfx(jgrp, acc):
        base = nb - (jgrp + 1) * 8
        hs = [hist[pl.ds((base + jj) * L, L)] for jj in range(8)]
        outs = [None] * 8
        for jj in reversed(range(8)):
            acc = acc + hs[jj]
            outs[jj] = acc
        for jj in range(8):
            hist[pl.ds((base + jj) * L, L)] = outs[jj]
        return acc
    lax.fori_loop(0, nb // 8, sfx, jnp.zeros((L,), jnp.int32))

    def stot(b):
        return jnp.sum(hist[pl.ds(b * L, L)])

    def bit(i, b):
        cand = b | (jnp.int32(1) << (7 - i))
        ok = (cand < nb) & (stot(cand) >= k)
        return lax.select(ok, cand, b)
    b = lax.fori_loop(0, 8, bit, jnp.int32(0))
    above = stot(b + 1)
    cnt_at = stot(b) - above
    return b, above, cnt_at


UNROLL = 8


def _hist_row(row, hist):
    lane = _lane()
    ones = jnp.ones((L,), jnp.int32)

    def body(i, c):
        base = i * (L * UNROLL)
        vs = [row[pl.ds(base + j * L, L)] for j in range(UNROLL)]
        bs = [lax.bitcast_convert_type(v, jnp.int32) & SIGN for v in vs]
        idxs = [((b >> 23) << 4) | lane for b in bs]
        for idx in idxs:
            plsc.addupdate_scatter(hist, [idx], ones)
        return c
    lax.fori_loop(0, NSL // UNROLL, body, 0)


def _compact_row(row, dst, bsel):
    lane = _lane()

    def body(i, off):
        base = i * (L * UNROLL)
        vs = [row[pl.ds(base + j * L, L)] for j in range(UNROLL)]
        bs = [lax.bitcast_convert_type(v, jnp.int32) & SIGN for v in vs]
        ms = [(b >> 23) == bsel for b in bs]
        poss = [plsc.cumsum(m.astype(jnp.int32)) for m in ms]
        pcs = [plsc.all_reduce_population_count(m) for m in ms]
        offs = [off]
        for pc in pcs:
            offs.append(offs[-1] + pc)
        for j in range(UNROLL):
            plsc.store_scatter(dst, [offs[j] + poss[j] - 1], bs[j],
                               mask=ms[j])
        return offs[UNROLL]
    off = lax.fori_loop(0, NSL // UNROLL, body, jnp.zeros((L,), jnp.int32))
    return jnp.squeeze(lax.slice(off, (0,), (1,)))


CU = 4


def _hist_cand(src, n, shift, bmask, hist):
    lane = _lane()
    ones = jnp.ones((L,), jnp.int32)
    nblk = (n + L * CU - 1) // (L * CU)

    def body(i, c):
        base = i * (L * CU)
        bss = [src[pl.ds(base + j * L, L)] for j in range(CU)]
        valids = [lane < (n - base - j * L) for j in range(CU)]
        idxs = [(((b >> shift) & bmask) << 4) | lane for b in bss]
        for j in range(CU):
            plsc.addupdate_scatter(hist, [idxs[j]], ones, mask=valids[j])
        return c
    lax.fori_loop(0, nblk, body, 0)


def _compact_cand(src, dst, n, shift, bmask, bsel, cap):
    lane = _lane()
    nblk = (n + L * CU - 1) // (L * CU)

    def body(i, off):
        base = i * (L * CU)
        bss = [src[pl.ds(base + j * L, L)] for j in range(CU)]
        ms = [(lane < (n - base - j * L))
              & (((b >> shift) & bmask) == bsel)
              for j, b in enumerate(bss)]
        poss = [plsc.cumsum(m.astype(jnp.int32)) for m in ms]
        pcs = [plsc.all_reduce_population_count(m) for m in ms]
        offs = [off]
        for pc in pcs:
            offs.append(offs[-1] + pc)
        for j in range(CU):
            idx = jnp.minimum(offs[j] + poss[j] - 1, jnp.int32(cap - 1))
            plsc.store_scatter(dst, [idx], bss[j], mask=ms[j])
        return offs[CU]
    off = lax.fori_loop(0, nblk, body, jnp.zeros((L,), jnp.int32))
    return jnp.minimum(jnp.squeeze(lax.slice(off, (0,), (1,))),
                       jnp.int32(cap))


def _mask_row(row, t, quota, tie_partial):
    zero = jnp.zeros((L,), jnp.float32)

    @pl.when(jnp.logical_not(tie_partial))
    def _():
        def body(i, c):
            base = i * (L * UNROLL)
            vs = [row[pl.ds(base + j * L, L)] for j in range(UNROLL)]
            bs = [lax.bitcast_convert_type(v, jnp.int32) & SIGN for v in vs]
            os_ = [jnp.where(b >= t, v, zero) for v, b in zip(vs, bs)]
            for j in range(UNROLL):
                row[pl.ds(base + j * L, L)] = os_[j]
            return c
        lax.fori_loop(0, NSL // UNROLL, body, 0)

    @pl.when(tie_partial)
    def _():
        def body(i, base):
            v = row[pl.ds(i * L, L)]
            bits = lax.bitcast_convert_type(v, jnp.int32) & SIGN
            eq = bits == t
            rank = base + plsc.cumsum(eq.astype(jnp.int32))
            keep = (bits > t) | (eq & (rank <= quota))
            row[pl.ds(i * L, L)] = jnp.where(keep, v, zero)
            return base + plsc.all_reduce_population_count(eq)
        lax.fori_loop(0, NSL, body, jnp.zeros((L,), jnp.int32))


_MESH = plsc.VectorSubcoreMesh(core_axis_name="c", subcore_axis_name="s")


@functools.partial(
    pl.kernel,
    mesh=_MESH,
    out_type=jax.ShapeDtypeStruct((R, C), jnp.float32),
    compiler_params=pltpu.CompilerParams(needs_layout_passes=False),
    scratch_types=[
        pltpu.VMEM((C,), jnp.float32),
        pltpu.VMEM((4224,), jnp.int32),
        pltpu.VMEM((C,), jnp.int32),
        pltpu.VMEM((CAND2_CAP,), jnp.int32),
    ],
)
def _sc_topk(x_hbm, out_hbm, row_v, hist_v, cand_v, cand2_v):
    wid = lax.axis_index("s") * 2 + lax.axis_index("c")

    def per_row(ri, c):
        r = wid * ROWS_PER_W + ri
        pltpu.sync_copy(x_hbm.at[r], row_v)

        _clear(hist_v, 257)
        _hist_row(row_v, hist_v)
        b1, above1, cnt1 = _select_bucket(hist_v, 256, jnp.int32(K))
        k1 = jnp.int32(K) - above1
        m1 = _compact_row(row_v, cand_v, b1)

        _clear(hist_v, 257)
        _hist_cand(cand_v, m1, 15, jnp.int32(0xFF), hist_v)
        b2, above2, cnt2 = _select_bucket(hist_v, 256, k1)
        k2 = k1 - above2
        m2 = _compact_cand(cand_v, cand2_v, m1, 15, jnp.int32(0xFF), b2,
                           CAND2_CAP)

        _clear(hist_v, 257)
        _hist_cand(cand2_v, m2, 7, jnp.int32(0xFF), hist_v)
        b3, above3, cnt3 = _select_bucket(hist_v, 256, k2)
        k3 = k2 - above3
        m3 = _compact_cand(cand2_v, cand_v, m2, 7, jnp.int32(0xFF), b3, C)

        _clear(hist_v, 129)
        _hist_cand(cand_v, m3, 0, jnp.int32(0x7F), hist_v)
        b4, above4, cnt4 = _select_bucket(hist_v, 128, k3)
        quota = k3 - above4

        t = (b1 << 23) | (b2 << 15) | (b3 << 7) | b4
        _mask_row(row_v, t, quota, quota < cnt4)
        pltpu.sync_copy(row_v, out_hbm.at[r])
        return c

    lax.fori_loop(0, ROWS_PER_W, per_row, 0)


def kernel(input_):
    return _sc_topk(input_)


# speedup vs baseline: 11.7444x; 1.0521x over previous
import functools

import jax
import jax.numpy as jnp
from jax import lax
from jax.experimental import pallas as pl
from jax.experimental.pallas import tpu as pltpu
from jax.experimental.pallas import tpu_sc as plsc

K = 64
R, C = 128, 32768
L = 16
NSL = C // L
NW = 32
ROWS_PER_W = R // NW
CAND2_CAP = 8192
SIGN = 0x7FFFFFFF


def _lane():
    return lax.broadcasted_iota(jnp.int32, (L,), 0)


def _clear(ref, nslices):
    def body(i, c):
        base = i * (L * 8)
        for j in range(8):
            ref[pl.ds(base + j * L, L)] = jnp.zeros((L,), jnp.int32)
        return c
    lax.fori_loop(0, (nslices + 7) // 8, body, 0)


def _select_bucket(hist, nb, k):
    def sfx(jgrp, acc):
        base = nb - (jgrp + 1) * 8
        hs = [hist[pl.ds((base + jj) * L, L)] for jj in range(8)]
        outs = [None] * 8
        for jj in reversed(range(8)):
            acc = acc + hs[jj]
            outs[jj] = acc
        for jj in range(8):
            hist[pl.ds((base + jj) * L, L)] = outs[jj]
        return acc
    lax.fori_loop(0, nb // 8, sfx, jnp.zeros((L,), jnp.int32))

    def stot(b):
        return jnp.sum(hist[pl.ds(b * L, L)])

    def bit(i, b):
        cand = b | (jnp.int32(1) << (7 - i))
        ok = (cand < nb) & (stot(cand) >= k)
        return lax.select(ok, cand, b)
    b = lax.fori_loop(0, 8, bit, jnp.int32(0))
    above = stot(b + 1)
    cnt_at = stot(b) - above
    return b, above, cnt_at


UNROLL = 8


def _hist_row(row, hist):
    lane = _lane()
    ones = jnp.ones((L,), jnp.int32)

    def body(i, c):
        base = i * (L * UNROLL)
        vs = [row[pl.ds(base + j * L, L)] for j in range(UNROLL)]
        bs = [lax.bitcast_convert_type(v, jnp.int32) & SIGN for v in vs]
        idxs = [((b >> 23) << 4) | lane for b in bs]
        for idx in idxs:
            plsc.addupdate_scatter(hist, [idx], ones)
        return c
    lax.fori_loop(0, NSL // UNROLL, body, 0)


def _compact_row(row, dst, bsel):
    lane = _lane()

    def body(i, off):
        base = i * (L * UNROLL)
        vs = [row[pl.ds(base + j * L, L)] for j in range(UNROLL)]
        bs = [lax.bitcast_convert_type(v, jnp.int32) & SIGN for v in vs]
        ms = [(b >> 23) == bsel for b in bs]
        poss = [plsc.cumsum(m.astype(jnp.int32)) for m in ms]
        pcs = [plsc.all_reduce_population_count(m) for m in ms]
        offs = [off]
        for pc in pcs:
            offs.append(offs[-1] + pc)
        for j in range(UNROLL):
            plsc.store_scatter(dst, [offs[j] + poss[j] - 1], bs[j],
                               mask=ms[j])
        return offs[UNROLL]
    off = lax.fori_loop(0, NSL // UNROLL, body, jnp.zeros((L,), jnp.int32))
    return jnp.squeeze(lax.slice(off, (0,), (1,)))


CU = 4


def _hist_cand(src, n, shift, bmask, hist):
    lane = _lane()
    ones = jnp.ones((L,), jnp.int32)
    nblk = (n + L * CU - 1) // (L * CU)

    def body(i, c):
        base = i * (L * CU)
        bss = [src[pl.ds(base + j * L, L)] for j in range(CU)]
        valids = [lane < (n - base - j * L) for j in range(CU)]
        idxs = [(((b >> shift) & bmask) << 4) | lane for b in bss]
        for j in range(CU):
            plsc.addupdate_scatter(hist, [idxs[j]], ones, mask=valids[j])
        return c
    lax.fori_loop(0, nblk, body, 0)


def _compact_cand(src, dst, n, shift, bmask, bsel, cap):
    lane = _lane()
    nblk = (n + L * CU - 1) // (L * CU)

    def body(i, off):
        base = i * (L * CU)
        bss = [src[pl.ds(base + j * L, L)] for j in range(CU)]
        ms = [(lane < (n - base - j * L))
              & (((b >> shift) & bmask) == bsel)
              for j, b in enumerate(bss)]
        poss = [plsc.cumsum(m.astype(jnp.int32)) for m in ms]
        pcs = [plsc.all_reduce_population_count(m) for m in ms]
        offs = [off]
        for pc in pcs:
            offs.append(offs[-1] + pc)
        for j in range(CU):
            idx = jnp.minimum(offs[j] + poss[j] - 1, jnp.int32(cap - 1))
            plsc.store_scatter(dst, [idx], bss[j], mask=ms[j])
        return offs[CU]
    off = lax.fori_loop(0, nblk, body, jnp.zeros((L,), jnp.int32))
    return jnp.minimum(jnp.squeeze(lax.slice(off, (0,), (1,))),
                       jnp.int32(cap))


def _mask_row(row, t, quota, tie_partial):
    zero = jnp.zeros((L,), jnp.float32)

    @pl.when(jnp.logical_not(tie_partial))
    def _():
        def body(i, c):
            base = i * (L * UNROLL)
            vs = [row[pl.ds(base + j * L, L)] for j in range(UNROLL)]
            bs = [lax.bitcast_convert_type(v, jnp.int32) & SIGN for v in vs]
            os_ = [jnp.where(b >= t, v, zero) for v, b in zip(vs, bs)]
            for j in range(UNROLL):
                row[pl.ds(base + j * L, L)] = os_[j]
            return c
        lax.fori_loop(0, NSL // UNROLL, body, 0)

    @pl.when(tie_partial)
    def _():
        def body(i, base):
            v = row[pl.ds(i * L, L)]
            bits = lax.bitcast_convert_type(v, jnp.int32) & SIGN
            eq = bits == t
            rank = base + plsc.cumsum(eq.astype(jnp.int32))
            keep = (bits > t) | (eq & (rank <= quota))
            row[pl.ds(i * L, L)] = jnp.where(keep, v, zero)
            return base + plsc.all_reduce_population_count(eq)
        lax.fori_loop(0, NSL, body, jnp.zeros((L,), jnp.int32))


_MESH = plsc.VectorSubcoreMesh(core_axis_name="c", subcore_axis_name="s")


@functools.partial(
    pl.kernel,
    mesh=_MESH,
    out_type=jax.ShapeDtypeStruct((R, C), jnp.float32),
    compiler_params=pltpu.CompilerParams(needs_layout_passes=False),
    scratch_types=[
        pltpu.VMEM((C,), jnp.float32),
        pltpu.VMEM((C,), jnp.float32),
        pltpu.VMEM((4224,), jnp.int32),
        pltpu.VMEM((C,), jnp.int32),
        pltpu.VMEM((CAND2_CAP,), jnp.int32),
        pltpu.SemaphoreType.DMA,
        pltpu.SemaphoreType.DMA,
        pltpu.SemaphoreType.DMA,
        pltpu.SemaphoreType.DMA,
    ],
)
def _sc_topk(x_hbm, out_hbm, row0_v, row1_v, hist_v, cand_v, cand2_v,
             si0, si1, so0, so1):
    wid = lax.axis_index("s") * 2 + lax.axis_index("c")
    bufs = (row0_v, row1_v)
    isems = (si0, si1)
    osems = (so0, so1)
    rows = [wid * ROWS_PER_W + ri for ri in range(ROWS_PER_W)]

    in_cp = [None] * ROWS_PER_W
    out_cp = [None] * ROWS_PER_W
    in_cp[0] = pltpu.async_copy(x_hbm.at[rows[0]], bufs[0], isems[0])

    for ri in range(ROWS_PER_W):
        cur = bufs[ri % 2]
        in_cp[ri].wait()

        _clear(hist_v, 257)
        _hist_row(cur, hist_v)

        if ri + 1 < ROWS_PER_W:
            if ri >= 1:
                out_cp[ri - 1].wait()
            in_cp[ri + 1] = pltpu.async_copy(
                x_hbm.at[rows[ri + 1]], bufs[(ri + 1) % 2],
                isems[(ri + 1) % 2])

        b1, above1, cnt1 = _select_bucket(hist_v, 256, jnp.int32(K))
        k1 = jnp.int32(K) - above1
        m1 = _compact_row(cur, cand_v, b1)

        _clear(hist_v, 257)
        _hist_cand(cand_v, m1, 15, jnp.int32(0xFF), hist_v)
        b2, above2, cnt2 = _select_bucket(hist_v, 256, k1)
        k2 = k1 - above2
        m2 = _compact_cand(cand_v, cand2_v, m1, 15, jnp.int32(0xFF), b2,
                           CAND2_CAP)

        _clear(hist_v, 257)
        _hist_cand(cand2_v, m2, 7, jnp.int32(0xFF), hist_v)
        b3, above3, cnt3 = _select_bucket(hist_v, 256, k2)
        k3 = k2 - above3
        m3 = _compact_cand(cand2_v, cand_v, m2, 7, jnp.int32(0xFF), b3, C)

        _clear(hist_v, 129)
        _hist_cand(cand_v, m3, 0, jnp.int32(0x7F), hist_v)
        b4, above4, cnt4 = _select_bucket(hist_v, 128, k3)
        quota = k3 - above4

        t = (b1 << 23) | (b2 << 15) | (b3 << 7) | b4
        _mask_row(cur, t, quota, quota < cnt4)
        out_cp[ri] = pltpu.async_copy(cur, out_hbm.at[rows[ri]],
                                      osems[ri % 2])

    out_cp[ROWS_PER_W - 2].wait()
    out_cp[ROWS_PER_W - 1].wait()


def kernel(input_):
    return _sc_topk(input_)


# speedup vs baseline: 13.3306x; 1.1351x over previous
import functools

import jax
import jax.numpy as jnp
from jax import lax
from jax.experimental import pallas as pl
from jax.experimental.pallas import tpu as pltpu
from jax.experimental.pallas import tpu_sc as plsc

K = 64
R, C = 128, 32768
L = 16
NSL = C // L
NW = 32
ROWS_PER_W = R // NW
CAND2_CAP = 8192
SIGN = 0x7FFFFFFF


def _lane():
    return lax.broadcasted_iota(jnp.int32, (L,), 0)


def _clear(ref, nslices):
    zero = jnp.zeros((L,), jnp.int32)

    @plsc.parallel_loop(0, ((nslices + 7) // 8) * 8, unroll=8)
    def _(i):
        ref[pl.ds(i * L, L)] = zero


def _select_bucket(hist, nb, k):
    @plsc.parallel_loop(0, nb, unroll=8, carry=jnp.zeros((L,), jnp.int32))
    def _sfx(j, acc):
        b = nb - 1 - j
        acc = acc + hist[pl.ds(b * L, L)]
        hist[pl.ds(b * L, L)] = acc
        return acc

    def stot(b):
        return jnp.sum(hist[pl.ds(b * L, L)])

    def bit(i, b):
        cand = b | (jnp.int32(1) << (7 - i))
        ok = (cand < nb) & (stot(cand) >= k)
        return lax.select(ok, cand, b)
    b = lax.fori_loop(0, 8, bit, jnp.int32(0))
    above = stot(b + 1)
    cnt_at = stot(b) - above
    return b, above, cnt_at


UNROLL = 8


def _hist_row(row, hist):
    lane = _lane()
    ones = jnp.ones((L,), jnp.int32)

    @plsc.parallel_loop(0, NSL, unroll=UNROLL)
    def _(i):
        v = row[pl.ds(i * L, L)]
        bits = lax.bitcast_convert_type(v, jnp.int32) & SIGN
        idx = ((bits >> 23) << 4) | lane
        plsc.addupdate_scatter(hist, [idx], ones)


def _compact_row(row, dst, bsel):
    lane = _lane()

    @plsc.parallel_loop(0, NSL, unroll=UNROLL,
                        carry=jnp.zeros((L,), jnp.int32))
    def off(i, off):
        v = row[pl.ds(i * L, L)]
        bits = lax.bitcast_convert_type(v, jnp.int32) & SIGN
        m = (bits >> 23) == bsel
        pos = plsc.cumsum(m.astype(jnp.int32))
        plsc.store_scatter(dst, [off + pos - 1], bits, mask=m)
        return off + plsc.all_reduce_population_count(m)
    return jnp.squeeze(lax.slice(off, (0,), (1,)))


CU = 4


def _hist_cand(src, n, shift, bmask, hist):
    lane = _lane()
    ones = jnp.ones((L,), jnp.int32)
    nsl = (n + L - 1) // L

    @plsc.parallel_loop(0, nsl, unroll=CU)
    def _(i):
        valid = lane < (n - i * L)
        bits = src[pl.ds(i * L, L)]
        idx = (((bits >> shift) & bmask) << 4) | lane
        plsc.addupdate_scatter(hist, [idx], ones, mask=valid)


def _compact_cand(src, dst, n, shift, bmask, bsel, cap):
    lane = _lane()
    nsl = (n + L - 1) // L

    @plsc.parallel_loop(0, nsl, unroll=CU,
                        carry=jnp.zeros((L,), jnp.int32))
    def off(i, off):
        valid = lane < (n - i * L)
        bits = src[pl.ds(i * L, L)]
        m = valid & (((bits >> shift) & bmask) == bsel)
        pos = plsc.cumsum(m.astype(jnp.int32))
        idx = jnp.minimum(off + pos - 1, jnp.int32(cap - 1))
        plsc.store_scatter(dst, [idx], bits, mask=m)
        return off + plsc.all_reduce_population_count(m)
    return jnp.minimum(jnp.squeeze(lax.slice(off, (0,), (1,))),
                       jnp.int32(cap))


def _mask_row(row, t, quota, tie_partial):
    zero = jnp.zeros((L,), jnp.float32)

    @pl.when(jnp.logical_not(tie_partial))
    def _():
        @plsc.parallel_loop(0, NSL, unroll=UNROLL)
        def _(i):
            v = row[pl.ds(i * L, L)]
            bits = lax.bitcast_convert_type(v, jnp.int32) & SIGN
            row[pl.ds(i * L, L)] = jnp.where(bits >= t, v, zero)

    @pl.when(tie_partial)
    def _():
        @plsc.parallel_loop(0, NSL, unroll=2,
                            carry=jnp.zeros((L,), jnp.int32))
        def _(i, base):
            v = row[pl.ds(i * L, L)]
            bits = lax.bitcast_convert_type(v, jnp.int32) & SIGN
            eq = bits == t
            rank = base + plsc.cumsum(eq.astype(jnp.int32))
            keep = (bits > t) | (eq & (rank <= quota))
            row[pl.ds(i * L, L)] = jnp.where(keep, v, zero)
            return base + plsc.all_reduce_population_count(eq)


_MESH = plsc.VectorSubcoreMesh(core_axis_name="c", subcore_axis_name="s")


@functools.partial(
    pl.kernel,
    mesh=_MESH,
    out_type=jax.ShapeDtypeStruct((R, C), jnp.float32),
    compiler_params=pltpu.CompilerParams(needs_layout_passes=False),
    scratch_types=[
        pltpu.VMEM((C,), jnp.float32),
        pltpu.VMEM((C,), jnp.float32),
        pltpu.VMEM((4224,), jnp.int32),
        pltpu.VMEM((C,), jnp.int32),
        pltpu.VMEM((CAND2_CAP,), jnp.int32),
        pltpu.SemaphoreType.DMA,
        pltpu.SemaphoreType.DMA,
        pltpu.SemaphoreType.DMA,
        pltpu.SemaphoreType.DMA,
    ],
)
def _sc_topk(x_hbm, out_hbm, row0_v, row1_v, hist_v, cand_v, cand2_v,
             si0, si1, so0, so1):
    wid = lax.axis_index("s") * 2 + lax.axis_index("c")
    bufs = (row0_v, row1_v)
    isems = (si0, si1)
    osems = (so0, so1)
    rows = [wid * ROWS_PER_W + ri for ri in range(ROWS_PER_W)]

    in_cp = [None] * ROWS_PER_W
    out_cp = [None] * ROWS_PER_W
    in_cp[0] = pltpu.async_copy(x_hbm.at[rows[0]], bufs[0], isems[0])

    for ri in range(ROWS_PER_W):
        cur = bufs[ri % 2]
        in_cp[ri].wait()

        _clear(hist_v, 257)
        _hist_row(cur, hist_v)

        if ri + 1 < ROWS_PER_W:
            if ri >= 1:
                out_cp[ri - 1].wait()
            in_cp[ri + 1] = pltpu.async_copy(
                x_hbm.at[rows[ri + 1]], bufs[(ri + 1) % 2],
                isems[(ri + 1) % 2])

        b1, above1, cnt1 = _select_bucket(hist_v, 256, jnp.int32(K))
        k1 = jnp.int32(K) - above1
        m1 = _compact_row(cur, cand_v, b1)

        _clear(hist_v, 257)
        _hist_cand(cand_v, m1, 15, jnp.int32(0xFF), hist_v)
        b2, above2, cnt2 = _select_bucket(hist_v, 256, k1)
        k2 = k1 - above2
        m2 = _compact_cand(cand_v, cand2_v, m1, 15, jnp.int32(0xFF), b2,
                           CAND2_CAP)

        _clear(hist_v, 257)
        _hist_cand(cand2_v, m2, 7, jnp.int32(0xFF), hist_v)
        b3, above3, cnt3 = _select_bucket(hist_v, 256, k2)
        k3 = k2 - above3
        m3 = _compact_cand(cand2_v, cand_v, m2, 7, jnp.int32(0xFF), b3, C)

        _clear(hist_v, 129)
        _hist_cand(cand_v, m3, 0, jnp.int32(0x7F), hist_v)
        b4, above4, cnt4 = _select_bucket(hist_v, 128, k3)
        quota = k3 - above4

        t = (b1 << 23) | (b2 << 15) | (b3 << 7) | b4
        _mask_row(cur, t, quota, quota < cnt4)
        out_cp[ri] = pltpu.async_copy(cur, out_hbm.at[rows[ri]],
                                      osems[ri % 2])

    out_cp[ROWS_PER_W - 2].wait()
    out_cp[ROWS_PER_W - 1].wait()


def kernel(input_):
    return _sc_topk(input_)
